# baseline (speedup 1.0000x reference)
import functools

import jax
import jax.numpy as jnp
from jax import lax
from jax.experimental import pallas as pl
from jax.experimental.pallas import tpu as pltpu
from jax.experimental.pallas import tpu_sc as plsc

VOCAB = 1000000
EMBED = 64
HIDDEN = 128
OUT = 1
BATCH = 4096
HIST = 200

_NC = 2
_NS = 16
_NW = _NC * _NS
_HALF = 100
_HALVES_PER_ROW = HIST // _HALF
_NHALVES = BATCH * _HALVES_PER_ROW
_HPW = _NHALVES // _NW
_BPW = BATCH // _NW


@functools.lru_cache(maxsize=1)
def _get_sc_pool():
    mesh = plsc.VectorSubcoreMesh(core_axis_name="c", subcore_axis_name="s")

    @functools.partial(
        pl.kernel,
        out_type=jax.ShapeDtypeStruct((BATCH, EMBED), jnp.float32),
        mesh=mesh,
        compiler_params=pltpu.CompilerParams(use_tc_tiling_on_sc=False),
        scratch_types=[
            pltpu.VMEM((_HPW, _HALF), jnp.int32),
            pltpu.VMEM((_HALF, EMBED), jnp.float32),
            pltpu.VMEM((_HALF, EMBED), jnp.float32),
            pltpu.VMEM((_BPW, EMBED), jnp.float32),
            pltpu.SemaphoreType.DMA,
            pltpu.SemaphoreType.DMA,
        ],
    )
    def _sc_pool(idx_hbm, table_hbm, out_hbm, idx_v, buf0, buf1, acc_v,
                 sem0, sem1):
        wid = lax.axis_index("s") * _NC + lax.axis_index("c")
        hbase = wid * _HPW
        obase = wid * _BPW

        pltpu.sync_copy(idx_hbm.at[pl.ds(hbase, _HPW)], idx_v)

        bufs = (buf0, buf1)
        sems = (sem0, sem1)

        pltpu.make_async_copy(table_hbm.at[idx_v.at[0]], buf0, sem0).start()

        zero = jnp.zeros((16,), jnp.float32)

        def _reduce_chunk(buf, carry):
            def rbody(j, a):
                a0, a1, a2, a3 = a
                a0 = a0 + buf[j, pl.ds(0, 16)]
                a1 = a1 + buf[j, pl.ds(16, 16)]
                a2 = a2 + buf[j, pl.ds(32, 16)]
                a3 = a3 + buf[j, pl.ds(48, 16)]
                return (a0, a1, a2, a3)
            return lax.fori_loop(0, _HALF, rbody, carry)

        def body(b, carry_unused):
            acc = (zero, zero, zero, zero)
            for h in range(_HALVES_PER_ROW):
                g = _HALVES_PER_ROW * b + h
                nxt = g + 1

                @pl.when(nxt < _HPW)
                def _():
                    pltpu.make_async_copy(
                        table_hbm.at[idx_v.at[nxt]],
                        bufs[1 - h], sems[1 - h]).start()

                pltpu.make_async_copy(
                    table_hbm.at[idx_v.at[g]], bufs[h], sems[h]).wait()

                acc = _reduce_chunk(bufs[h], acc)

            a0, a1, a2, a3 = acc
            acc_v[b, pl.ds(0, 16)] = a0
            acc_v[b, pl.ds(16, 16)] = a1
            acc_v[b, pl.ds(32, 16)] = a2
            acc_v[b, pl.ds(48, 16)] = a3
            return carry_unused

        lax.fori_loop(0, _BPW, body, 0)

        pltpu.sync_copy(acc_v, out_hbm.at[pl.ds(obase, _BPW)])

    return _sc_pool


def _mlp_body(x_ref, w1_ref, b1_ref, w2_ref, b2_ref, o_ref):
    x = x_ref[...] * (1.0 / HIST)
    h = jnp.dot(x, w1_ref[...], preferred_element_type=jnp.float32)
    h = jnp.maximum(h + b1_ref[...], 0.0)
    o = jnp.dot(h, w2_ref[...], preferred_element_type=jnp.float32)
    o = o + b2_ref[...]
    o_ref[...] = 1.0 / (1.0 + jnp.exp(-o))


def kernel(inputs, table, W1, b1, W2, b2):
    idx = inputs.reshape(_NHALVES, _HALF).astype(jnp.int32)
    pooled = _get_sc_pool()(idx, table)
    return pl.pallas_call(
        _mlp_body,
        out_shape=jax.ShapeDtypeStruct((BATCH, OUT), jnp.float32),
    )(pooled, W1, b1.reshape(1, HIDDEN), W2, b2.reshape(1, OUT))


# speedup vs baseline: 1.1335x; 1.1335x over previous
import functools

import jax
import jax.numpy as jnp
from jax import lax
from jax.experimental import pallas as pl
from jax.experimental.pallas import tpu as pltpu
from jax.experimental.pallas import tpu_sc as plsc

VOCAB = 1000000
EMBED = 64
HIDDEN = 128
OUT = 1
BATCH = 4096
HIST = 200

_NC = 2
_NS = 16
_NW = _NC * _NS
_HALF = 100
_HALVES_PER_ROW = HIST // _HALF
_NHALVES = BATCH * _HALVES_PER_ROW
_HPW = _NHALVES // _NW
_BPW = BATCH // _NW


_NBUF = 8


@functools.lru_cache(maxsize=1)
def _get_sc_pool():
    mesh = plsc.VectorSubcoreMesh(core_axis_name="c", subcore_axis_name="s")

    @functools.partial(
        pl.kernel,
        out_type=jax.ShapeDtypeStruct((BATCH, EMBED), jnp.float32),
        mesh=mesh,
        compiler_params=pltpu.CompilerParams(use_tc_tiling_on_sc=False),
        scratch_types=(
            [pltpu.VMEM((_HPW, _HALF), jnp.int32)]
            + [pltpu.VMEM((_HALF, EMBED), jnp.float32)
               for _ in range(_NBUF)]
            + [pltpu.VMEM((_BPW, EMBED), jnp.float32)]
            + [pltpu.SemaphoreType.DMA for _ in range(_NBUF)]
        ),
    )
    def _sc_pool(idx_hbm, table_hbm, out_hbm, idx_v, *rest):
        bufs = rest[:_NBUF]
        acc_v = rest[_NBUF]
        sems = rest[_NBUF + 1:]

        wid = lax.axis_index("s") * _NC + lax.axis_index("c")
        hbase = wid * _HPW
        obase = wid * _BPW

        pltpu.sync_copy(idx_hbm.at[pl.ds(hbase, _HPW)], idx_v)

        for i in range(_NBUF):
            pltpu.make_async_copy(
                table_hbm.at[idx_v.at[i]], bufs[i], sems[i]).start()

        zero = jnp.zeros((16,), jnp.float32)

        def _reduce_chunk(buf, acc):
            def rbody(j, a):
                a = list(a)
                base = j * 4
                for r in range(4):
                    for k in range(4):
                        slot = k + 4 * (r % 2)
                        a[slot] = a[slot] + buf[base + r, pl.ds(16 * k, 16)]
                return tuple(a)
            return lax.fori_loop(0, _HALF // 4, rbody, acc)

        def outer(t, carry_unused):
            acc = (zero,) * 8
            for i in range(_NBUF):
                g = t * _NBUF + i
                pltpu.make_async_copy(
                    table_hbm.at[idx_v.at[g]], bufs[i], sems[i]).wait()

                if i % 2 == 0:
                    acc = _reduce_chunk(bufs[i], (zero,) * 8)
                else:
                    acc = _reduce_chunk(bufs[i], acc)
                    row = t * (_NBUF // 2) + i // 2
                    acc_v[row, pl.ds(0, 16)] = acc[0] + acc[4]
                    acc_v[row, pl.ds(16, 16)] = acc[1] + acc[5]
                    acc_v[row, pl.ds(32, 16)] = acc[2] + acc[6]
                    acc_v[row, pl.ds(48, 16)] = acc[3] + acc[7]

                nxt = g + _NBUF

                @pl.when(nxt < _HPW)
                def _():
                    pltpu.make_async_copy(
                        table_hbm.at[idx_v.at[nxt]], bufs[i], sems[i]).start()
            return carry_unused

        lax.fori_loop(0, _HPW // _NBUF, outer, 0)

        pltpu.sync_copy(acc_v, out_hbm.at[pl.ds(obase, _BPW)])

    return _sc_pool


def _mlp_body(x_ref, w1_ref, b1_ref, w2_ref, b2_ref, o_ref):
    x = x_ref[...] * (1.0 / HIST)
    h = jnp.dot(x, w1_ref[...], preferred_element_type=jnp.float32)
    h = jnp.maximum(h + b1_ref[...], 0.0)
    o = jnp.dot(h, w2_ref[...], preferred_element_type=jnp.float32)
    o = o + b2_ref[...]
    o_ref[...] = 1.0 / (1.0 + jnp.exp(-o))


def kernel(inputs, table, W1, b1, W2, b2):
    idx = inputs.reshape(_NHALVES, _HALF).astype(jnp.int32)
    pooled = _get_sc_pool()(idx, table)
    return pl.pallas_call(
        _mlp_body,
        out_shape=jax.ShapeDtypeStruct((BATCH, OUT), jnp.float32),
    )(pooled, W1, b1.reshape(1, HIDDEN), W2, b2.reshape(1, OUT))


# speedup vs baseline: 2.3276x; 2.0535x over previous
import functools

import jax
import jax.numpy as jnp
from jax import lax
from jax.experimental import pallas as pl
from jax.experimental.pallas import tpu as pltpu
from jax.experimental.pallas import tpu_sc as plsc

VOCAB = 1000000
EMBED = 64
HIDDEN = 128
OUT = 1
BATCH = 4096
HIST = 200

_NC = 2
_NS = 16
_NW = _NC * _NS
_HALF = 100
_HALVES_PER_ROW = HIST // _HALF
_NHALVES = BATCH * _HALVES_PER_ROW
_HPW = _NHALVES // _NW
_BPW = BATCH // _NW


_NBUF = 8

_PACKN = 16384
_PACKH = _PACKN // 2
_NBLK = (VOCAB + _PACKN - 1) // _PACKN
_T2ROWS = _NBLK * _PACKH


def _pack_body(in_ref, o_ref):
    t = in_ref[...].T
    o_ref[:, 0:64] = t[0:_PACKH, :]
    o_ref[:, 64:128] = t[_PACKH:_PACKN, :]


def _pack_table(table):
    return pl.pallas_call(
        _pack_body,
        grid=(_NBLK,),
        in_specs=[pl.BlockSpec((EMBED, _PACKN), lambda i: (0, i))],
        out_specs=pl.BlockSpec((_PACKH, 128), lambda i: (i, 0)),
        out_shape=jax.ShapeDtypeStruct((_T2ROWS, 128), jnp.float32),
    )(table.T)


def _remap_indices(r):
    blk = r >> 14
    j = r & (_PACKN - 1)
    return (((blk << 13) + (j & (_PACKH - 1))) << 1) + (j >> 13)


@functools.lru_cache(maxsize=1)
def _get_sc_pool():
    mesh = plsc.VectorSubcoreMesh(core_axis_name="c", subcore_axis_name="s")

    @functools.partial(
        pl.kernel,
        out_type=jax.ShapeDtypeStruct((BATCH, EMBED), jnp.float32),
        mesh=mesh,
        compiler_params=pltpu.CompilerParams(use_tc_tiling_on_sc=False),
        scratch_types=(
            [pltpu.VMEM((_HPW, _HALF), jnp.int32)]
            + [pltpu.VMEM((_HALF, EMBED), jnp.float32)
               for _ in range(_NBUF)]
            + [pltpu.VMEM((_BPW, EMBED), jnp.float32)]
            + [pltpu.SemaphoreType.DMA for _ in range(_NBUF)]
        ),
    )
    def _sc_pool(idx_hbm, table_hbm, out_hbm, idx_v, *rest):
        bufs = rest[:_NBUF]
        acc_v = rest[_NBUF]
        sems = rest[_NBUF + 1:]

        wid = lax.axis_index("s") * _NC + lax.axis_index("c")
        hbase = wid * _HPW
        obase = wid * _BPW

        pltpu.sync_copy(idx_hbm.at[pl.ds(hbase, _HPW)], idx_v)

        for i in range(_NBUF):
            pltpu.make_async_copy(
                table_hbm.at[idx_v.at[i]], bufs[i], sems[i]).start()

        zero = jnp.zeros((16,), jnp.float32)

        def _reduce_chunk(buf, acc):
            def rbody(j, a):
                a = list(a)
                base = j * 4
                for r in range(4):
                    for k in range(4):
                        slot = k + 4 * (r % 2)
                        a[slot] = a[slot] + buf[base + r, pl.ds(16 * k, 16)]
                return tuple(a)
            return lax.fori_loop(0, _HALF // 4, rbody, acc)

        def outer(t, carry_unused):
            acc = (zero,) * 8
            for i in range(_NBUF):
                g = t * _NBUF + i
                pltpu.make_async_copy(
                    table_hbm.at[idx_v.at[g]], bufs[i], sems[i]).wait()

                if i % 2 == 0:
                    acc = _reduce_chunk(bufs[i], (zero,) * 8)
                else:
                    acc = _reduce_chunk(bufs[i], acc)
                    row = t * (_NBUF // 2) + i // 2
                    acc_v[row, pl.ds(0, 16)] = acc[0] + acc[4]
                    acc_v[row, pl.ds(16, 16)] = acc[1] + acc[5]
                    acc_v[row, pl.ds(32, 16)] = acc[2] + acc[6]
                    acc_v[row, pl.ds(48, 16)] = acc[3] + acc[7]

                nxt = g + _NBUF

                @pl.when(nxt < _HPW)
                def _():
                    pltpu.make_async_copy(
                        table_hbm.at[idx_v.at[nxt]], bufs[i], sems[i]).start()
            return carry_unused

        lax.fori_loop(0, _HPW // _NBUF, outer, 0)

        pltpu.sync_copy(acc_v, out_hbm.at[pl.ds(obase, _BPW)])

    return _sc_pool


def _mlp_body(x_ref, w1_ref, b1_ref, w2_ref, b2_ref, o_ref):
    x = x_ref[...] * (1.0 / HIST)
    h = jnp.dot(x, w1_ref[...], preferred_element_type=jnp.float32)
    h = jnp.maximum(h + b1_ref[...], 0.0)
    o = jnp.dot(h, w2_ref[...], preferred_element_type=jnp.float32)
    o = o + b2_ref[...]
    o_ref[...] = 1.0 / (1.0 + jnp.exp(-o))


def kernel(inputs, table, W1, b1, W2, b2):
    t2 = _pack_table(table).reshape(2 * _T2ROWS, EMBED)
    idx = _remap_indices(inputs.astype(jnp.int32)).reshape(_NHALVES, _HALF)
    pooled = _get_sc_pool()(idx, t2)
    return pl.pallas_call(
        _mlp_body,
        out_shape=jax.ShapeDtypeStruct((BATCH, OUT), jnp.float32),
    )(pooled, W1, b1.reshape(1, HIDDEN), W2, b2.reshape(1, OUT))


# speedup vs baseline: 2.8197x; 1.2114x over previous
import functools

import jax
import jax.numpy as jnp
from jax import lax
from jax.experimental import pallas as pl
from jax.experimental.pallas import tpu as pltpu
from jax.experimental.pallas import tpu_sc as plsc

VOCAB = 1000000
EMBED = 64
HIDDEN = 128
OUT = 1
BATCH = 4096
HIST = 200

_NC = 2
_NS = 16
_NW = _NC * _NS
_HALF = 100
_HALVES_PER_ROW = HIST // _HALF
_NHALVES = BATCH * _HALVES_PER_ROW
_HPW = _NHALVES // _NW
_BPW = BATCH // _NW


_NBUF = 8

_PACKN = 16384
_PACKH = _PACKN // 2
_NBLK = (VOCAB + _PACKN - 1) // _PACKN
_T2ROWS = _NBLK * _PACKH


def _pack_body(in_ref, o_ref):
    s = jnp.concatenate(
        [in_ref[:, 0:_PACKH], in_ref[:, _PACKH:_PACKN]], axis=0)
    o_ref[...] = s.T


def _pack_table(table):
    return pl.pallas_call(
        _pack_body,
        grid=(_NBLK,),
        in_specs=[pl.BlockSpec((EMBED, _PACKN), lambda i: (0, i))],
        out_specs=pl.BlockSpec((_PACKH, 128), lambda i: (i, 0)),
        out_shape=jax.ShapeDtypeStruct((_T2ROWS, 128), jnp.float32),
    )(table.T)


def _remap_indices(r):
    blk = r >> 14
    j = r & (_PACKN - 1)
    return (((blk << 13) + (j & (_PACKH - 1))) << 1) + (j >> 13)


@functools.lru_cache(maxsize=1)
def _get_sc_pool():
    mesh = plsc.VectorSubcoreMesh(core_axis_name="c", subcore_axis_name="s")

    @functools.partial(
        pl.kernel,
        out_type=jax.ShapeDtypeStruct((BATCH, EMBED), jnp.float32),
        mesh=mesh,
        compiler_params=pltpu.CompilerParams(use_tc_tiling_on_sc=False),
        scratch_types=(
            [pltpu.VMEM((_HPW, _HALF), jnp.int32)]
            + [pltpu.VMEM((_HALF, EMBED), jnp.float32)
               for _ in range(_NBUF)]
            + [pltpu.VMEM((_BPW, EMBED), jnp.float32)]
            + [pltpu.SemaphoreType.DMA for _ in range(_NBUF)]
        ),
    )
    def _sc_pool(idx_hbm, table_hbm, out_hbm, idx_v, *rest):
        bufs = rest[:_NBUF]
        acc_v = rest[_NBUF]
        sems = rest[_NBUF + 1:]

        wid = lax.axis_index("s") * _NC + lax.axis_index("c")
        hbase = wid * _HPW
        obase = wid * _BPW

        pltpu.sync_copy(idx_hbm.at[pl.ds(hbase, _HPW)], idx_v)

        for i in range(_NBUF):
            pltpu.make_async_copy(
                table_hbm.at[idx_v.at[i]], bufs[i], sems[i]).start()

        zero = jnp.zeros((16,), jnp.float32)

        def _reduce_chunk(buf, acc):
            def rbody(j, a):
                a = list(a)
                base = j * 4
                for r in range(4):
                    for k in range(4):
                        slot = k + 4 * (r % 2)
                        a[slot] = a[slot] + buf[base + r, pl.ds(16 * k, 16)]
                return tuple(a)
            return lax.fori_loop(0, _HALF // 4, rbody, acc)

        def outer(t, carry_unused):
            acc = (zero,) * 8
            for i in range(_NBUF):
                g = t * _NBUF + i
                pltpu.make_async_copy(
                    table_hbm.at[idx_v.at[g]], bufs[i], sems[i]).wait()

                if i % 2 == 0:
                    acc = _reduce_chunk(bufs[i], (zero,) * 8)
                else:
                    acc = _reduce_chunk(bufs[i], acc)
                    row = t * (_NBUF // 2) + i // 2
                    acc_v[row, pl.ds(0, 16)] = acc[0] + acc[4]
                    acc_v[row, pl.ds(16, 16)] = acc[1] + acc[5]
                    acc_v[row, pl.ds(32, 16)] = acc[2] + acc[6]
                    acc_v[row, pl.ds(48, 16)] = acc[3] + acc[7]

                nxt = g + _NBUF

                @pl.when(nxt < _HPW)
                def _():
                    pltpu.make_async_copy(
                        table_hbm.at[idx_v.at[nxt]], bufs[i], sems[i]).start()
            return carry_unused

        lax.fori_loop(0, _HPW // _NBUF, outer, 0)

        pltpu.sync_copy(acc_v, out_hbm.at[pl.ds(obase, _BPW)])

    return _sc_pool


def _mlp_body(x_ref, w1_ref, b1_ref, w2_ref, b2_ref, o_ref):
    x = x_ref[...] * (1.0 / HIST)
    h = jnp.dot(x, w1_ref[...], preferred_element_type=jnp.float32)
    h = jnp.maximum(h + b1_ref[...], 0.0)
    o = jnp.dot(h, w2_ref[...], preferred_element_type=jnp.float32)
    o = o + b2_ref[...]
    o_ref[...] = 1.0 / (1.0 + jnp.exp(-o))


def kernel(inputs, table, W1, b1, W2, b2):
    t2 = _pack_table(table).reshape(2 * _T2ROWS, EMBED)
    idx = _remap_indices(inputs.astype(jnp.int32)).reshape(_NHALVES, _HALF)
    pooled = _get_sc_pool()(idx, t2)
    return pl.pallas_call(
        _mlp_body,
        out_shape=jax.ShapeDtypeStruct((BATCH, OUT), jnp.float32),
    )(pooled, W1, b1.reshape(1, HIDDEN), W2, b2.reshape(1, OUT))


# speedup vs baseline: 3.6650x; 1.2998x over previous
import functools

import jax
import jax.numpy as jnp
from jax import lax
from jax.experimental import pallas as pl
from jax.experimental.pallas import tpu as pltpu
from jax.experimental.pallas import tpu_sc as plsc

VOCAB = 1000000
EMBED = 64
HIDDEN = 128
OUT = 1
BATCH = 4096
HIST = 200

_NC = 2
_NS = 16
_NW = _NC * _NS
_HALF = 100
_HALVES_PER_ROW = HIST // _HALF
_NHALVES = BATCH * _HALVES_PER_ROW
_HPW = _NHALVES // _NW
_BPW = BATCH // _NW


_NBUF = 8

_PACKN = 16384
_PACKH = _PACKN // 2
_NBLK = (VOCAB + _PACKN - 1) // _PACKN
_T2ROWS = _NBLK * _PACKH


_QTR = _PACKN // 4


def _pack_body(in_ref, o_ref):
    s = jnp.concatenate(
        [in_ref[:, q * _QTR:(q + 1) * _QTR] for q in range(4)], axis=0)
    u = jax.lax.bitcast_convert_type(
        s.astype(jnp.bfloat16), jnp.uint16).astype(jnp.uint32)
    z = jnp.concatenate(
        [u[64 * q:64 * q + 32, :] | (u[64 * q + 32:64 * q + 64, :] << 16)
         for q in range(4)], axis=0)
    o_ref[...] = z.T


def _pack_table(table):
    return pl.pallas_call(
        _pack_body,
        grid=(_NBLK,),
        in_specs=[pl.BlockSpec((EMBED, _PACKN), lambda i: (0, i))],
        out_specs=pl.BlockSpec((_QTR, 128), lambda i: (i, 0)),
        out_shape=jax.ShapeDtypeStruct((_NBLK * _QTR, 128), jnp.uint32),
    )(table.T)


def _remap_indices(r):
    blk = r >> 14
    j = r & (_PACKN - 1)
    return (((blk << 12) + (j & (_QTR - 1))) << 2) + (j >> 12)


_DIM_PERM = (
    tuple(range(0, 16)) + tuple(range(32, 48))
    + tuple(range(16, 32)) + tuple(range(48, 64))
)


@functools.lru_cache(maxsize=1)
def _get_sc_pool():
    mesh = plsc.VectorSubcoreMesh(core_axis_name="c", subcore_axis_name="s")

    @functools.partial(
        pl.kernel,
        out_type=jax.ShapeDtypeStruct((BATCH, EMBED), jnp.float32),
        mesh=mesh,
        compiler_params=pltpu.CompilerParams(
            use_tc_tiling_on_sc=False, needs_layout_passes=False),
        scratch_types=(
            [pltpu.VMEM((_HPW, _HALF), jnp.int32)]
            + [pltpu.VMEM((_HALF, EMBED // 2), jnp.uint32)
               for _ in range(_NBUF)]
            + [pltpu.VMEM((_BPW, EMBED), jnp.float32)]
            + [pltpu.SemaphoreType.DMA for _ in range(_NBUF)]
        ),
    )
    def _sc_pool(idx_hbm, table_hbm, out_hbm, idx_v, *rest):
        bufs = rest[:_NBUF]
        acc_v = rest[_NBUF]
        sems = rest[_NBUF + 1:]

        wid = lax.axis_index("s") * _NC + lax.axis_index("c")
        hbase = wid * _HPW
        obase = wid * _BPW

        pltpu.sync_copy(idx_hbm.at[pl.ds(hbase, _HPW)], idx_v)

        for i in range(_NBUF):
            pltpu.make_async_copy(
                table_hbm.at[idx_v.at[i]], bufs[i], sems[i]).start()

        zero = jnp.zeros((16,), jnp.float32)

        def _reduce_chunk(buf, acc):
            def rbody(j, a):
                a = list(a)
                base = j * 4
                for r in range(4):
                    s = 4 * (r % 2)
                    w0 = plsc.bitcast(buf[base + r, pl.ds(0, 16)],
                                      jnp.bfloat16)
                    w1 = plsc.bitcast(buf[base + r, pl.ds(16, 16)],
                                      jnp.bfloat16)
                    g0, g1 = plsc.unpack(
                        w0, format=plsc.PackFormat.INTERLEAVED)
                    g2, g3 = plsc.unpack(
                        w1, format=plsc.PackFormat.INTERLEAVED)
                    a[s + 0] = a[s + 0] + g0
                    a[s + 1] = a[s + 1] + g1
                    a[s + 2] = a[s + 2] + g2
                    a[s + 3] = a[s + 3] + g3
                return tuple(a)
            return lax.fori_loop(0, _HALF // 4, rbody, acc)

        def outer(t, carry_unused):
            acc = (zero,) * 8
            for i in range(_NBUF):
                g = t * _NBUF + i
                pltpu.make_async_copy(
                    table_hbm.at[idx_v.at[g]], bufs[i], sems[i]).wait()

                if i % 2 == 0:
                    acc = _reduce_chunk(bufs[i], (zero,) * 8)
                else:
                    acc = _reduce_chunk(bufs[i], acc)
                    row = t * (_NBUF // 2) + i // 2
                    acc_v[row, pl.ds(0, 16)] = acc[0] + acc[4]
                    acc_v[row, pl.ds(16, 16)] = acc[1] + acc[5]
                    acc_v[row, pl.ds(32, 16)] = acc[2] + acc[6]
                    acc_v[row, pl.ds(48, 16)] = acc[3] + acc[7]

                nxt = g + _NBUF

                @pl.when(nxt < _HPW)
                def _():
                    pltpu.make_async_copy(
                        table_hbm.at[idx_v.at[nxt]], bufs[i], sems[i]).start()
            return carry_unused

        lax.fori_loop(0, _HPW // _NBUF, outer, 0)

        pltpu.sync_copy(acc_v, out_hbm.at[pl.ds(obase, _BPW)])

    return _sc_pool


def _mlp_body(x_ref, w1_ref, b1_ref, w2_ref, b2_ref, o_ref):
    x = x_ref[...] * (1.0 / HIST)
    h = jnp.dot(x, w1_ref[...], preferred_element_type=jnp.float32)
    h = jnp.maximum(h + b1_ref[...], 0.0)
    o = jnp.dot(h, w2_ref[...], preferred_element_type=jnp.float32)
    o = o + b2_ref[...]
    o_ref[...] = 1.0 / (1.0 + jnp.exp(-o))


def kernel(inputs, table, W1, b1, W2, b2):
    t2 = _pack_table(table).reshape(_NBLK * _PACKN, EMBED // 2)
    idx = _remap_indices(inputs.astype(jnp.int32)).reshape(_NHALVES, _HALF)
    pooled = _get_sc_pool()(idx, t2)
    w1p = W1[jnp.array(_DIM_PERM), :]
    return pl.pallas_call(
        _mlp_body,
        out_shape=jax.ShapeDtypeStruct((BATCH, OUT), jnp.float32),
    )(pooled, w1p, b1.reshape(1, HIDDEN), W2, b2.reshape(1, OUT))


# speedup vs baseline: 3.7827x; 1.0321x over previous
import functools

import jax
import jax.numpy as jnp
from jax import lax
from jax.experimental import pallas as pl
from jax.experimental.pallas import tpu as pltpu
from jax.experimental.pallas import tpu_sc as plsc

VOCAB = 1000000
EMBED = 64
HIDDEN = 128
OUT = 1
BATCH = 4096
HIST = 200

_NC = 2
_NS = 16
_NW = _NC * _NS
_HALF = 100
_HALVES_PER_ROW = HIST // _HALF
_NHALVES = BATCH * _HALVES_PER_ROW
_HPW = _NHALVES // _NW
_BPW = BATCH // _NW


_NBUF = 16

_PACKN = 32768
_PACKH = _PACKN // 2
_NBLK = (VOCAB + _PACKN - 1) // _PACKN
_T2ROWS = _NBLK * _PACKH


_QTR = _PACKN // 4


def _pack_body(in_ref, o_ref):
    s = jnp.concatenate(
        [in_ref[:, q * _QTR:(q + 1) * _QTR] for q in range(4)], axis=0)
    u = jax.lax.bitcast_convert_type(
        s.astype(jnp.bfloat16), jnp.uint16).astype(jnp.uint32)
    z = jnp.concatenate(
        [u[64 * q:64 * q + 32, :] | (u[64 * q + 32:64 * q + 64, :] << 16)
         for q in range(4)], axis=0)
    o_ref[...] = z.T


def _pack_table(table):
    return pl.pallas_call(
        _pack_body,
        grid=(_NBLK,),
        in_specs=[pl.BlockSpec((EMBED, _PACKN), lambda i: (0, i))],
        out_specs=pl.BlockSpec((_QTR, 128), lambda i: (i, 0)),
        out_shape=jax.ShapeDtypeStruct((_NBLK * _QTR, 128), jnp.uint32),
    )(table.T)


def _remap_indices(r):
    blk = r >> 15
    j = r & (_PACKN - 1)
    return (((blk << 13) + (j & (_QTR - 1))) << 2) + (j >> 13)


_DIM_PERM = (
    tuple(range(0, 16)) + tuple(range(32, 48))
    + tuple(range(16, 32)) + tuple(range(48, 64))
)


@functools.lru_cache(maxsize=1)
def _get_sc_pool():
    mesh = plsc.VectorSubcoreMesh(core_axis_name="c", subcore_axis_name="s")

    @functools.partial(
        pl.kernel,
        out_type=jax.ShapeDtypeStruct((BATCH, EMBED), jnp.float32),
        mesh=mesh,
        compiler_params=pltpu.CompilerParams(
            use_tc_tiling_on_sc=False, needs_layout_passes=False),
        scratch_types=(
            [pltpu.VMEM((_HPW, _HALF), jnp.int32)]
            + [pltpu.VMEM((_HALF, EMBED // 2), jnp.uint32)
               for _ in range(_NBUF)]
            + [pltpu.VMEM((_BPW, EMBED), jnp.float32)]
            + [pltpu.SemaphoreType.DMA for _ in range(_NBUF)]
        ),
    )
    def _sc_pool(idx_hbm, table_hbm, out_hbm, idx_v, *rest):
        bufs = rest[:_NBUF]
        acc_v = rest[_NBUF]
        sems = rest[_NBUF + 1:]

        wid = lax.axis_index("s") * _NC + lax.axis_index("c")
        hbase = wid * _HPW
        obase = wid * _BPW

        pltpu.sync_copy(idx_hbm.at[pl.ds(hbase, _HPW)], idx_v)

        for i in range(_NBUF):
            pltpu.make_async_copy(
                table_hbm.at[idx_v.at[i]], bufs[i], sems[i]).start()

        zero = jnp.zeros((16,), jnp.float32)

        def _reduce_chunk(buf, acc):
            def rbody(j, a):
                a = list(a)
                base = j * 4
                for r in range(4):
                    s = 4 * (r % 2)
                    w0 = plsc.bitcast(buf[base + r, pl.ds(0, 16)],
                                      jnp.bfloat16)
                    w1 = plsc.bitcast(buf[base + r, pl.ds(16, 16)],
                                      jnp.bfloat16)
                    g0, g1 = plsc.unpack(
                        w0, format=plsc.PackFormat.INTERLEAVED)
                    g2, g3 = plsc.unpack(
                        w1, format=plsc.PackFormat.INTERLEAVED)
                    a[s + 0] = a[s + 0] + g0
                    a[s + 1] = a[s + 1] + g1
                    a[s + 2] = a[s + 2] + g2
                    a[s + 3] = a[s + 3] + g3
                return tuple(a)
            return lax.fori_loop(0, _HALF // 4, rbody, acc)

        def outer(t, carry_unused):
            acc = (zero,) * 8
            for i in range(_NBUF):
                g = t * _NBUF + i
                pltpu.make_async_copy(
                    table_hbm.at[idx_v.at[g]], bufs[i], sems[i]).wait()

                if i % 2 == 0:
                    acc = _reduce_chunk(bufs[i], (zero,) * 8)
                else:
                    acc = _reduce_chunk(bufs[i], acc)
                    row = t * (_NBUF // 2) + i // 2
                    acc_v[row, pl.ds(0, 16)] = acc[0] + acc[4]
                    acc_v[row, pl.ds(16, 16)] = acc[1] + acc[5]
                    acc_v[row, pl.ds(32, 16)] = acc[2] + acc[6]
                    acc_v[row, pl.ds(48, 16)] = acc[3] + acc[7]

                nxt = g + _NBUF

                @pl.when(nxt < _HPW)
                def _():
                    pltpu.make_async_copy(
                        table_hbm.at[idx_v.at[nxt]], bufs[i], sems[i]).start()
            return carry_unused

        lax.fori_loop(0, _HPW // _NBUF, outer, 0)

        pltpu.sync_copy(acc_v, out_hbm.at[pl.ds(obase, _BPW)])

    return _sc_pool


def _mlp_body(x_ref, w1_ref, b1_ref, w2_ref, b2_ref, o_ref):
    x = x_ref[...] * (1.0 / HIST)
    h = jnp.dot(x, w1_ref[...], preferred_element_type=jnp.float32)
    h = jnp.maximum(h + b1_ref[...], 0.0)
    o = jnp.dot(h, w2_ref[...], preferred_element_type=jnp.float32)
    o = o + b2_ref[...]
    o_ref[...] = 1.0 / (1.0 + jnp.exp(-o))


def kernel(inputs, table, W1, b1, W2, b2):
    t2 = _pack_table(table).reshape(_NBLK * _PACKN, EMBED // 2)
    idx = _remap_indices(inputs.astype(jnp.int32)).reshape(_NHALVES, _HALF)
    pooled = _get_sc_pool()(idx, t2)
    w1p = W1[jnp.array(_DIM_PERM), :]
    return pl.pallas_call(
        _mlp_body,
        out_shape=jax.ShapeDtypeStruct((BATCH, OUT), jnp.float32),
    )(pooled, w1p, b1.reshape(1, HIDDEN), W2, b2.reshape(1, OUT))
